# baseline (speedup 1.0000x reference)
import functools

import jax
import jax.numpy as jnp
from jax import lax
from jax.experimental import pallas as pl
from jax.experimental.pallas import tpu as pltpu
from jax.experimental.pallas import tpu_sc as plsc

D_MODEL = 1024
NUM_CORES = 2
NUM_SUBCORES = 16
NUM_WORKERS = NUM_CORES * NUM_SUBCORES


@functools.partial(jax.jit, static_argnames=())
def _gather_rows(pe, idx_flat):
    n = idx_flat.shape[0]
    b_per_w = n // NUM_WORKERS
    mesh = plsc.VectorSubcoreMesh(core_axis_name="c", subcore_axis_name="s")

    @functools.partial(
        pl.kernel,
        mesh=mesh,
        out_type=jax.ShapeDtypeStruct((n, D_MODEL), jnp.float32),
        scratch_types=[
            pltpu.VMEM((b_per_w,), jnp.int32),
            pltpu.SemaphoreType.DMA,
        ],
    )
    def k(table_hbm, idx_hbm, out_hbm, idx_v, sem):
        wid = lax.axis_index("s") * NUM_CORES + lax.axis_index("c")
        base = wid * b_per_w
        pltpu.sync_copy(idx_hbm.at[pl.ds(base, b_per_w)], idx_v)

        @pl.loop(0, b_per_w)
        def _(r):
            row = idx_v[pl.ds(r, 1)][0]
            pltpu.async_copy(
                table_hbm.at[pl.ds(row, 1)], out_hbm.at[pl.ds(base + r, 1)], sem
            )

        @pl.loop(0, b_per_w)
        def _(r):
            pltpu.make_async_copy(
                table_hbm.at[pl.ds(0, 1)], out_hbm.at[pl.ds(base, 1)], sem
            ).wait()

    return k(pe, idx_flat)


def kernel(pe, indexes):
    b, s = indexes.shape
    idx_flat = indexes.astype(jnp.int32).reshape(b * s)
    out = _gather_rows(pe, idx_flat)
    return out.reshape(b, s, D_MODEL)


# speedup vs baseline: 36.2623x; 36.2623x over previous
import functools

import jax
import jax.numpy as jnp
from jax import lax
from jax.experimental import pallas as pl
from jax.experimental.pallas import tpu as pltpu
from jax.experimental.pallas import tpu_sc as plsc

D_MODEL = 1024
NUM_CORES = 2
NUM_SUBCORES = 16
NUM_WORKERS = NUM_CORES * NUM_SUBCORES
CHUNK = 16
NBUF = 4
NSP = 2


@functools.partial(jax.jit, static_argnames=())
def _gather_rows(pe, idx_flat):
    n = idx_flat.shape[0]
    b_per_w = n // NUM_WORKERS
    n_chunks = b_per_w // CHUNK
    mesh = plsc.VectorSubcoreMesh(core_axis_name="c", subcore_axis_name="s")

    @functools.partial(
        pl.kernel,
        mesh=mesh,
        out_type=jax.ShapeDtypeStruct((n, D_MODEL), jnp.float32),
        scratch_types=[
            pltpu.VMEM((b_per_w,), jnp.int32),
            pltpu.VMEM_SHARED((NUM_SUBCORES * NSP * CHUNK, D_MODEL), jnp.float32),
        ]
        + [pltpu.VMEM((CHUNK, D_MODEL), jnp.float32)] * NBUF
        + [pltpu.SemaphoreType.DMA] * (NBUF + NSP),
    )
    def k(table_hbm, idx_hbm, out_hbm, idx_v, spm, *bufs_and_sems):
        bufs = bufs_and_sems[:NBUF]
        gsems = bufs_and_sems[NBUF : 2 * NBUF]
        wsems = bufs_and_sems[2 * NBUF :]
        sid = lax.axis_index("s")
        spms = [
            spm.at[pl.ds((sid * NSP + t) * CHUNK, CHUNK)] for t in range(NSP)
        ]
        wid = lax.axis_index("s") * NUM_CORES + lax.axis_index("c")
        base = wid * b_per_w
        pltpu.sync_copy(idx_hbm.at[pl.ds(base, b_per_w)], idx_v)

        def gather_start(ci, s):
            pltpu.async_copy(
                table_hbm.at[idx_v.at[pl.ds(ci * CHUNK, CHUNK)]], bufs[s], gsems[s]
            )

        def gather_wait(s):
            pltpu.make_async_copy(
                table_hbm.at[idx_v.at[pl.ds(0, CHUNK)]], bufs[s], gsems[s]
            ).wait()

        def write_start(ci, s):
            pltpu.async_copy(
                spms[s], out_hbm.at[pl.ds(base + ci * CHUNK, CHUNK)], wsems[s]
            )

        def write_drain(s):
            pltpu.make_async_copy(
                spms[s], out_hbm.at[pl.ds(base, CHUNK)], wsems[s]
            ).wait()

        gather_start(0, 0)
        gather_start(1, 1)

        @pl.loop(0, n_chunks, step=NBUF)
        def _(j):
            for s in range(NBUF):
                c = j + s
                s2 = (s + 2) % NBUF
                sp = s % NSP
                gather_wait(s)

                @pl.when(c >= NSP)
                def _(sp=sp):
                    write_drain(sp)

                pltpu.sync_copy(bufs[s], spms[sp])
                write_start(c, sp)

                @pl.when(c + 2 < n_chunks)
                def _(c=c, s2=s2):
                    gather_start(c + 2, s2)

        for t in range(NSP):
            write_drain(t)

    return k(pe, idx_flat)


def kernel(pe, indexes):
    b, s = indexes.shape
    idx_flat = indexes.astype(jnp.int32).reshape(b * s)
    out = _gather_rows(pe, idx_flat)
    return out.reshape(b, s, D_MODEL)


# speedup vs baseline: 36.3732x; 1.0031x over previous
import functools

import jax
import jax.numpy as jnp
from jax import lax
from jax.experimental import pallas as pl
from jax.experimental.pallas import tpu as pltpu
from jax.experimental.pallas import tpu_sc as plsc

D_MODEL = 1024
NUM_CORES = 2
NUM_SUBCORES = 16
NUM_WORKERS = NUM_CORES * NUM_SUBCORES
CHUNK = 16
NBUF = 4


@functools.partial(jax.jit, static_argnames=())
def _gather_rows(pe, idx_flat):
    n = idx_flat.shape[0]
    b_per_w = n // NUM_WORKERS
    n_chunks = b_per_w // CHUNK
    mesh = plsc.VectorSubcoreMesh(core_axis_name="c", subcore_axis_name="s")

    @functools.partial(
        pl.kernel,
        mesh=mesh,
        out_type=jax.ShapeDtypeStruct((n, D_MODEL), jnp.float32),
        scratch_types=[
            pltpu.VMEM((b_per_w,), jnp.int32),
        ]
        + [pltpu.VMEM((CHUNK, D_MODEL), jnp.float32)] * NBUF
        + [pltpu.SemaphoreType.DMA] * (2 * NBUF),
    )
    def k(table_hbm, idx_hbm, out_hbm, idx_v, *bufs_and_sems):
        bufs = bufs_and_sems[:NBUF]
        gsems = bufs_and_sems[NBUF : 2 * NBUF]
        wsems = bufs_and_sems[2 * NBUF :]
        wid = lax.axis_index("s") * NUM_CORES + lax.axis_index("c")
        base = wid * b_per_w
        pltpu.sync_copy(idx_hbm.at[pl.ds(base, b_per_w)], idx_v)

        def gather_start(ci, s):
            pltpu.async_copy(
                table_hbm.at[idx_v.at[pl.ds(ci * CHUNK, CHUNK)]], bufs[s], gsems[s]
            )

        def gather_wait(s):
            pltpu.make_async_copy(
                table_hbm.at[idx_v.at[pl.ds(0, CHUNK)]], bufs[s], gsems[s]
            ).wait()

        def write_start(ci, s):
            pltpu.async_copy(
                bufs[s], out_hbm.at[pl.ds(base + ci * CHUNK, CHUNK)], wsems[s]
            )

        def write_drain(s):
            pltpu.make_async_copy(
                bufs[s], out_hbm.at[pl.ds(base, CHUNK)], wsems[s]
            ).wait()

        gather_start(0, 0)
        gather_start(1, 1)

        @pl.loop(0, n_chunks, step=NBUF)
        def _(j):
            for s in range(NBUF):
                c = j + s
                s2 = (s + 2) % NBUF
                gather_wait(s)
                write_start(c, s)

                @pl.when(c + 2 < n_chunks)
                def _(c=c, s2=s2):
                    @pl.when(c >= 2)
                    def _():
                        write_drain(s2)

                    gather_start(c + 2, s2)

        for s in range(NBUF):
            write_drain(s)

    return k(pe, idx_flat)


def kernel(pe, indexes):
    b, s = indexes.shape
    idx_flat = indexes.astype(jnp.int32).reshape(b * s)
    out = _gather_rows(pe, idx_flat)
    return out.reshape(b, s, D_MODEL)
